# baseline (speedup 1.0000x reference)
import jax
import jax.numpy as jnp
from jax.experimental import pallas as pl
from jax.experimental.pallas import tpu as pltpu

B, S, R, D, M = 32, 512, 1024, 768, 64
BB = 4


def _score_one(ctx, cand, w):
    logits = jax.lax.dot_general(w, ctx, (((1,), (1,)), ((), ())),
                                 preferred_element_type=jnp.float32)
    e = jnp.exp(logits)
    s = jnp.sum(e, axis=-1, keepdims=True)
    embs_u = jnp.dot(e, ctx, preferred_element_type=jnp.float32)
    lt = jax.lax.dot_general(embs_u, cand, (((1,), (1,)), ((), ())),
                             preferred_element_type=jnp.float32) / s
    lm = jnp.max(lt, axis=0, keepdims=True)
    el = jnp.exp(lt - lm)
    return jnp.sum(el * lt, axis=0) / jnp.sum(el, axis=0)


def _poly_kernel(ctx_ref, cand_ref, w_ref, out_ref):
    w = w_ref[...]
    for k in range(BB):
        out_ref[0, k] = _score_one(ctx_ref[k], cand_ref[k], w)


def kernel(ctx_out, cand_emb, poly_code_weight):
    out3 = pl.pallas_call(
        _poly_kernel,
        grid=(B // BB,),
        in_specs=[
            pl.BlockSpec((BB, S, D), lambda b: (b, 0, 0)),
            pl.BlockSpec((BB, R, D), lambda b: (b, 0, 0)),
            pl.BlockSpec((M, D), lambda b: (0, 0)),
        ],
        out_specs=pl.BlockSpec((1, BB, R), lambda b: (b, 0, 0)),
        out_shape=jax.ShapeDtypeStruct((B // BB, BB, R), jnp.float32),
        compiler_params=pltpu.CompilerParams(
            dimension_semantics=("parallel",),
            vmem_limit_bytes=110 * 1024 * 1024),
    )(ctx_out, cand_emb, poly_code_weight)
    return out3.reshape(B, R)


# speedup vs baseline: 1.0033x; 1.0033x over previous
import jax
import jax.numpy as jnp
from jax.experimental import pallas as pl
from jax.experimental.pallas import tpu as pltpu

B, S, R, D, M = 32, 512, 1024, 768, 64
BB = 4


def _score_one(ctx, cand, w):
    logits = jax.lax.dot_general(w, ctx, (((1,), (1,)), ((), ())),
                                 preferred_element_type=jnp.float32)
    e = jnp.exp(logits)
    s = jnp.sum(e, axis=-1, keepdims=True)
    embs = jnp.dot(e, ctx, preferred_element_type=jnp.float32) / s
    lt = jax.lax.dot_general(embs, cand, (((1,), (1,)), ((), ())),
                             preferred_element_type=jnp.float32)
    lm = jnp.max(lt, axis=0, keepdims=True)
    el = jnp.exp(lt - lm)
    return jnp.sum(el * lt, axis=0) / jnp.sum(el, axis=0)


def _poly_kernel(ctx_ref, cand_ref, w_ref, out_ref):
    w = w_ref[...]
    for k in range(BB):
        out_ref[0, k] = _score_one(ctx_ref[k], cand_ref[k], w)


def kernel(ctx_out, cand_emb, poly_code_weight):
    out3 = pl.pallas_call(
        _poly_kernel,
        grid=(B // BB,),
        in_specs=[
            pl.BlockSpec((BB, S, D), lambda b: (b, 0, 0)),
            pl.BlockSpec((BB, R, D), lambda b: (b, 0, 0)),
            pl.BlockSpec((M, D), lambda b: (0, 0)),
        ],
        out_specs=pl.BlockSpec((1, BB, R), lambda b: (b, 0, 0)),
        out_shape=jax.ShapeDtypeStruct((B // BB, BB, R), jnp.float32),
        compiler_params=pltpu.CompilerParams(
            dimension_semantics=("parallel",),
            vmem_limit_bytes=110 * 1024 * 1024),
    )(ctx_out, cand_emb, poly_code_weight)
    return out3.reshape(B, R)
